# baseline (speedup 1.0000x reference)
import jax
import jax.numpy as jnp
import numpy as np
from jax import lax
from jax.experimental import pallas as pl
from jax.experimental.pallas import tpu as pltpu
from jax.experimental.pallas import tpu_sc as plsc

B, C, HX, HY, H = 8, 96, 224, 224, 2
ROWS = B * C
OX, OY = HX // 2, HY // 2
NPOOL = OX * OY
ROW_ELEMS = HX * HY * H
QB_ELEMS = ROW_ELEMS // 4
STRIP = 2 * HY * H
OUT_SZ = 100
KTH = 100

NC, NS = 2, 16
NW = NC * NS
RPW = ROWS // NW

INT_MIN = np.int32(-(2 ** 31))
_S1 = 1024
_S2 = 128


def _lane_gather(v, idx):
    return lax.gather(
        v, idx[:, None],
        lax.GatherDimensionNumbers(offset_dims=(), collapsed_slice_dims=(0,),
                                   start_index_map=(0,)),
        (1,), mode=lax.GatherScatterMode.PROMISE_IN_BOUNDS)


def _splat(s):
    return jnp.zeros((16,), jnp.int32) + s


def _sc_kernel(x_hbm, ids_hbm, pairs_hbm,
               xs_v, uu_v, ww_v, hh_v, sm_v, sp_v, su_v, sp2_v,
               oid_v, opr_v):
    wkr = lax.axis_index("s") * NC + lax.axis_index("c")
    lanes = lax.iota(jnp.int32, 16)
    rot = [((lanes + k) & 15) for k in range(16)]
    imin_v = jnp.full((16,), INT_MIN, jnp.int32)

    uu_v[pl.ds(NPOOL, 16)] = imin_v

    def row_body(r, _):
        r0 = wkr * RPW + r

        def qb_body(qb, acc):
            pltpu.sync_copy(x_hbm.at[r0, pl.ds(qb * QB_ELEMS, QB_ELEMS)],
                            xs_v)

            def strip_body(t, acc):
                base = t * STRIP
                i1 = qb * 28 + t
                cmx = imin_v
                for cc in range(7):
                    jl = cc * 16 + lanes
                    A = base + 4 * jl + 1
                    g0 = plsc.load_gather(xs_v, [A])
                    g1 = plsc.load_gather(xs_v, [A + 2])
                    g2 = plsc.load_gather(xs_v, [A + 448])
                    g3 = plsc.load_gather(xs_v, [A + 450])
                    best = g0
                    off = jnp.zeros((16,), jnp.int32)
                    for g, o in ((g1, 2), (g2, 448), (g3, 450)):
                        m = g > best
                        best = jnp.where(m, g, best)
                        off = jnp.where(m, jnp.int32(o), off)
                    di = jnp.where(off >= 448, jnp.int32(1), jnp.int32(0))
                    dj = (off & 2) >> 1
                    wid = (2 * i1 + di) * HY + 2 * jl + dj
                    h0 = plsc.load_gather(xs_v, [A + off - 1])
                    bits = plsc.bitcast(best, jnp.int32)
                    u = jnp.where(bits < 0, bits ^ jnp.int32(0x7FFFFFFF),
                                  bits)
                    pos = i1 * OY + cc * 16
                    uu_v[pl.ds(pos, 16)] = u
                    ww_v[pl.ds(pos, 16)] = wid
                    hh_v[pl.ds(pos, 16)] = h0
                    cmx = jnp.maximum(cmx, u)
                mx = jnp.max(cmx)
                acc = jnp.where(lanes == (i1 & 15), _splat(mx), acc)
                sm_v[pl.ds((i1 >> 4) * 16, 16)] = acc
                return acc

            return lax.fori_loop(0, 28, strip_body, acc)

        lax.fori_loop(0, 4, qb_body, imin_v)

        def t1_body(i, thr):
            cand = thr | (jnp.int32(1) << (31 - i))
            ts = _splat(cand ^ INT_MIN)
            cnt = jnp.int32(0)
            for k in range(7):
                sk = sm_v[pl.ds(k * 16, 16)]
                cnt = cnt + plsc.all_reduce_population_count(sk >= ts)[0]
            return jnp.where(cnt >= KTH, cand, thr)

        thr1 = lax.fori_loop(0, 16, t1_body, jnp.int32(0))
        tl1 = _splat(thr1 ^ INT_MIN)

        def filt(c, wp):
            uc = uu_v[pl.ds(c * 16, 16)]
            m = uc >= tl1
            wp_c = jnp.minimum(wp, _S1)
            cs = plsc.cumsum(jnp.where(m, jnp.int32(1), jnp.int32(0)))
            tgt = wp_c + cs - 1
            plsc.store_scatter(sp_v, [tgt], c * 16 + lanes, mask=m)
            plsc.store_scatter(su_v, [tgt], uc, mask=m)
            return wp + plsc.all_reduce_population_count(m)[0]

        s1 = lax.fori_loop(0, NPOOL // 16, filt, jnp.int32(0))
        s1 = jnp.minimum(s1, _S1)
        sp_v[pl.ds(s1, 16)] = _splat(NPOOL)
        su_v[pl.ds(s1, 16)] = imin_v
        nb1 = (s1 + 15) >> 4

        def t2_body(i, thr):
            cand = thr | (jnp.int32(1) << (31 - i))
            ts = _splat(cand ^ INT_MIN)

            def cnt_body(cb, acc):
                uS = su_v[pl.ds(cb * 16, 16)]
                return acc + plsc.all_reduce_population_count(uS >= ts)[0]

            cnt = lax.fori_loop(0, nb1, cnt_body, jnp.int32(0))
            return jnp.where(cnt >= KTH, cand, thr)

        thr2 = lax.fori_loop(0, 32, t2_body, jnp.int32(0))
        tl2 = _splat(thr2 ^ INT_MIN)

        def filt2(cb, wp):
            uS = su_v[pl.ds(cb * 16, 16)]
            pS = sp_v[pl.ds(cb * 16, 16)]
            m = uS >= tl2
            wp_c = jnp.minimum(wp, _S2)
            cs = plsc.cumsum(jnp.where(m, jnp.int32(1), jnp.int32(0)))
            plsc.store_scatter(sp2_v, [wp_c + cs - 1], pS, mask=m)
            return wp + plsc.all_reduce_population_count(m)[0]

        s2 = lax.fori_loop(0, nb1, filt2, jnp.int32(0))
        s2 = jnp.minimum(s2, _S2)
        sp2_v[pl.ds(s2, 16)] = _splat(NPOOL)
        nb2 = (s2 + 15) >> 4

        def rank_a(a, _):
            pA = sp2_v[pl.ds(a * 16, 16)]
            uA = plsc.load_gather(uu_v, [pA])
            jA = a * 16 + lanes

            def rank_b(bq, acc):
                uB = plsc.load_gather(uu_v, [sp2_v[pl.ds(bq * 16, 16)]])
                for k in range(16):
                    uBr = _lane_gather(uB, rot[k])
                    jBr = bq * 16 + rot[k]
                    w = (uBr > uA) | ((uBr == uA) & (jBr < jA))
                    acc = acc + jnp.where(w, jnp.int32(1), jnp.int32(0))
                return acc

            rA = lax.fori_loop(0, nb2, rank_b, jnp.zeros((16,), jnp.int32))
            mk = rA < OUT_SZ
            widA = plsc.load_gather(ww_v, [pA])
            h0A = plsc.load_gather(hh_v, [pA])
            vA = plsc.bitcast(
                jnp.where(uA < 0, uA ^ jnp.int32(0x7FFFFFFF), uA),
                jnp.float32)
            plsc.store_scatter(oid_v, [rA], widA, mask=mk)
            plsc.store_scatter(opr_v, [2 * rA], h0A, mask=mk)
            plsc.store_scatter(opr_v, [2 * rA + 1], vA, mask=mk)
            return 0

        lax.fori_loop(0, nb2, rank_a, jnp.int32(0))
        pltpu.sync_copy(oid_v, ids_hbm.at[r0])
        pltpu.sync_copy(opr_v, pairs_hbm.at[r0])
        return 0

    lax.fori_loop(0, RPW, row_body, jnp.int32(0))


def kernel(x):
    b, c, hx, hy, h = x.shape
    x2d = x.reshape(ROWS, ROW_ELEMS)

    mesh = plsc.VectorSubcoreMesh(core_axis_name="c", subcore_axis_name="s",
                                  num_cores=NC, num_subcores=NS)
    ids, pairs = pl.kernel(
        _sc_kernel,
        out_type=[
            jax.ShapeDtypeStruct((ROWS, OX), jnp.int32),
            jax.ShapeDtypeStruct((ROWS, 2 * OX), jnp.float32),
        ],
        mesh=mesh,
        compiler_params=pltpu.CompilerParams(needs_layout_passes=False),
        scratch_types=[
            pltpu.VMEM((QB_ELEMS,), jnp.float32),
            pltpu.VMEM((NPOOL + 16,), jnp.int32),
            pltpu.VMEM((NPOOL + 16,), jnp.int32),
            pltpu.VMEM((NPOOL + 16,), jnp.float32),
            pltpu.VMEM((OX,), jnp.int32),
            pltpu.VMEM((_S1 + 16,), jnp.int32),
            pltpu.VMEM((_S1 + 16,), jnp.int32),
            pltpu.VMEM((_S2 + 16,), jnp.int32),
            pltpu.VMEM((OX,), jnp.int32),
            pltpu.VMEM((2 * OX,), jnp.float32),
        ],
    )(x2d)

    x_out = pairs.reshape(ROWS, OX, 2)[:, :OUT_SZ, :].reshape(
        b, c, OUT_SZ, 1, h)
    sorted_ids = ids[:, :OUT_SZ].reshape(b, c, OUT_SZ)
    return x_out, sorted_ids, hx, hy


# speedup vs baseline: 1.0388x; 1.0388x over previous
import jax
import jax.numpy as jnp
import numpy as np
from jax import lax
from jax.experimental import pallas as pl
from jax.experimental.pallas import tpu as pltpu
from jax.experimental.pallas import tpu_sc as plsc

B, C, HX, HY, H = 8, 96, 224, 224, 2
ROWS = B * C
OX, OY = HX // 2, HY // 2
NPOOL = OX * OY
ROW_ELEMS = HX * HY * H
QB_ELEMS = ROW_ELEMS // 4
STRIP = 2 * HY * H
OUT_SZ = 100
KTH = 100

NC, NS = 2, 16
NW = NC * NS
RPW = ROWS // NW

INT_MIN = np.int32(-(2 ** 31))
_S1 = 1024
_S2 = 128


def _lane_gather(v, idx):
    return lax.gather(
        v, idx[:, None],
        lax.GatherDimensionNumbers(offset_dims=(), collapsed_slice_dims=(0,),
                                   start_index_map=(0,)),
        (1,), mode=lax.GatherScatterMode.PROMISE_IN_BOUNDS)


def _splat(s):
    return jnp.zeros((16,), jnp.int32) + s


def _sc_kernel(x_hbm, ids_hbm, pairs_hbm,
               xs_v, xs2_v, sem0, sem1, uu_v, ww_v, hh_v, sm_v, sp_v, su_v,
               sp2_v, oid_v, opr_v):
    wkr = lax.axis_index("s") * NC + lax.axis_index("c")
    lanes = lax.iota(jnp.int32, 16)
    rot = [((lanes + k) & 15) for k in range(16)]
    imin_v = jnp.full((16,), INT_MIN, jnp.int32)

    uu_v[pl.ds(NPOOL, 16)] = imin_v

    def row_body(r, _):
        r0 = wkr * RPW + r

        bufs = (xs_v, xs2_v)
        sems = (sem0, sem1)
        pltpu.make_async_copy(x_hbm.at[r0, pl.ds(0, QB_ELEMS)],
                              xs_v, sem0).start()
        acc0 = imin_v
        for qb in range(4):
            cur = bufs[qb % 2]
            pltpu.make_async_copy(x_hbm.at[r0, pl.ds(qb * QB_ELEMS,
                                                     QB_ELEMS)],
                                  cur, sems[qb % 2]).wait()
            if qb < 3:
                pltpu.make_async_copy(
                    x_hbm.at[r0, pl.ds((qb + 1) * QB_ELEMS, QB_ELEMS)],
                    bufs[(qb + 1) % 2], sems[(qb + 1) % 2]).start()

            def strip_body(t, acc, qb=qb, cur=cur):
                base = t * STRIP
                i1 = qb * 28 + t
                cmx = imin_v
                for cc in range(7):
                    jl = cc * 16 + lanes
                    A = base + 4 * jl + 1
                    g0 = plsc.load_gather(cur, [A])
                    g1 = plsc.load_gather(cur, [A + 2])
                    g2 = plsc.load_gather(cur, [A + 448])
                    g3 = plsc.load_gather(cur, [A + 450])
                    best = g0
                    off = jnp.zeros((16,), jnp.int32)
                    for g, o in ((g1, 2), (g2, 448), (g3, 450)):
                        m = g > best
                        best = jnp.where(m, g, best)
                        off = jnp.where(m, jnp.int32(o), off)
                    di = jnp.where(off >= 448, jnp.int32(1), jnp.int32(0))
                    dj = (off & 2) >> 1
                    wid = (2 * i1 + di) * HY + 2 * jl + dj
                    h0 = plsc.load_gather(cur, [A + off - 1])
                    bits = plsc.bitcast(best, jnp.int32)
                    u = jnp.where(bits < 0, bits ^ jnp.int32(0x7FFFFFFF),
                                  bits)
                    pos = i1 * OY + cc * 16
                    uu_v[pl.ds(pos, 16)] = u
                    ww_v[pl.ds(pos, 16)] = wid
                    hh_v[pl.ds(pos, 16)] = h0
                    cmx = jnp.maximum(cmx, u)
                mx = jnp.max(cmx)
                acc = jnp.where(lanes == (i1 & 15), _splat(mx), acc)
                sm_v[pl.ds((i1 >> 4) * 16, 16)] = acc
                return acc

            acc0 = lax.fori_loop(0, 28, strip_body, acc0)

        def t1_body(i, thr):
            cand = thr | (jnp.int32(1) << (31 - i))
            ts = _splat(cand ^ INT_MIN)
            cnt = jnp.int32(0)
            for k in range(7):
                sk = sm_v[pl.ds(k * 16, 16)]
                cnt = cnt + plsc.all_reduce_population_count(sk >= ts)[0]
            return jnp.where(cnt >= KTH, cand, thr)

        thr1 = lax.fori_loop(0, 16, t1_body, jnp.int32(0))
        tl1 = _splat(thr1 ^ INT_MIN)

        def filt(c, wp):
            uc = uu_v[pl.ds(c * 16, 16)]
            m = uc >= tl1
            wp_c = jnp.minimum(wp, _S1)
            cs = plsc.cumsum(jnp.where(m, jnp.int32(1), jnp.int32(0)))
            tgt = wp_c + cs - 1
            plsc.store_scatter(sp_v, [tgt], c * 16 + lanes, mask=m)
            plsc.store_scatter(su_v, [tgt], uc, mask=m)
            return wp + plsc.all_reduce_population_count(m)[0]

        s1 = lax.fori_loop(0, NPOOL // 16, filt, jnp.int32(0))
        s1 = jnp.minimum(s1, _S1)
        sp_v[pl.ds(s1, 16)] = _splat(NPOOL)
        su_v[pl.ds(s1, 16)] = imin_v
        nb1 = (s1 + 15) >> 4

        def t2_body(i, thr):
            cand = thr | (jnp.int32(1) << (31 - i))
            ts = _splat(cand ^ INT_MIN)

            def cnt_body(cb, acc):
                uS = su_v[pl.ds(cb * 16, 16)]
                return acc + plsc.all_reduce_population_count(uS >= ts)[0]

            cnt = lax.fori_loop(0, nb1, cnt_body, jnp.int32(0))
            return jnp.where(cnt >= KTH, cand, thr)

        thr2 = lax.fori_loop(0, 32, t2_body, jnp.int32(0))
        tl2 = _splat(thr2 ^ INT_MIN)

        def filt2(cb, wp):
            uS = su_v[pl.ds(cb * 16, 16)]
            pS = sp_v[pl.ds(cb * 16, 16)]
            m = uS >= tl2
            wp_c = jnp.minimum(wp, _S2)
            cs = plsc.cumsum(jnp.where(m, jnp.int32(1), jnp.int32(0)))
            plsc.store_scatter(sp2_v, [wp_c + cs - 1], pS, mask=m)
            return wp + plsc.all_reduce_population_count(m)[0]

        s2 = lax.fori_loop(0, nb1, filt2, jnp.int32(0))
        s2 = jnp.minimum(s2, _S2)
        sp2_v[pl.ds(s2, 16)] = _splat(NPOOL)
        nb2 = (s2 + 15) >> 4

        def rank_a(a, _):
            pA = sp2_v[pl.ds(a * 16, 16)]
            uA = plsc.load_gather(uu_v, [pA])
            jA = a * 16 + lanes

            def rank_b(bq, acc):
                uB = plsc.load_gather(uu_v, [sp2_v[pl.ds(bq * 16, 16)]])
                for k in range(16):
                    uBr = _lane_gather(uB, rot[k])
                    jBr = bq * 16 + rot[k]
                    w = (uBr > uA) | ((uBr == uA) & (jBr < jA))
                    acc = acc + jnp.where(w, jnp.int32(1), jnp.int32(0))
                return acc

            rA = lax.fori_loop(0, nb2, rank_b, jnp.zeros((16,), jnp.int32))
            mk = rA < OUT_SZ
            widA = plsc.load_gather(ww_v, [pA])
            h0A = plsc.load_gather(hh_v, [pA])
            vA = plsc.bitcast(
                jnp.where(uA < 0, uA ^ jnp.int32(0x7FFFFFFF), uA),
                jnp.float32)
            plsc.store_scatter(oid_v, [rA], widA, mask=mk)
            plsc.store_scatter(opr_v, [2 * rA], h0A, mask=mk)
            plsc.store_scatter(opr_v, [2 * rA + 1], vA, mask=mk)
            return 0

        lax.fori_loop(0, nb2, rank_a, jnp.int32(0))
        pltpu.sync_copy(oid_v, ids_hbm.at[r0])
        pltpu.sync_copy(opr_v, pairs_hbm.at[r0])
        return 0

    lax.fori_loop(0, RPW, row_body, jnp.int32(0))


def kernel(x):
    b, c, hx, hy, h = x.shape
    x2d = x.reshape(ROWS, ROW_ELEMS)

    mesh = plsc.VectorSubcoreMesh(core_axis_name="c", subcore_axis_name="s",
                                  num_cores=NC, num_subcores=NS)
    ids, pairs = pl.kernel(
        _sc_kernel,
        out_type=[
            jax.ShapeDtypeStruct((ROWS, OX), jnp.int32),
            jax.ShapeDtypeStruct((ROWS, 2 * OX), jnp.float32),
        ],
        mesh=mesh,
        compiler_params=pltpu.CompilerParams(needs_layout_passes=False),
        scratch_types=[
            pltpu.VMEM((QB_ELEMS,), jnp.float32),
            pltpu.VMEM((QB_ELEMS,), jnp.float32),
            pltpu.SemaphoreType.DMA,
            pltpu.SemaphoreType.DMA,
            pltpu.VMEM((NPOOL + 16,), jnp.int32),
            pltpu.VMEM((NPOOL + 16,), jnp.int32),
            pltpu.VMEM((NPOOL + 16,), jnp.float32),
            pltpu.VMEM((OX,), jnp.int32),
            pltpu.VMEM((_S1 + 16,), jnp.int32),
            pltpu.VMEM((_S1 + 16,), jnp.int32),
            pltpu.VMEM((_S2 + 16,), jnp.int32),
            pltpu.VMEM((OX,), jnp.int32),
            pltpu.VMEM((2 * OX,), jnp.float32),
        ],
    )(x2d)

    x_out = pairs.reshape(ROWS, OX, 2)[:, :OUT_SZ, :].reshape(
        b, c, OUT_SZ, 1, h)
    sorted_ids = ids[:, :OUT_SZ].reshape(b, c, OUT_SZ)
    return x_out, sorted_ids, hx, hy
